# Initial kernel scaffold; baseline (speedup 1.0000x reference)
import jax
import jax.numpy as jnp
from jax.experimental import pallas as pl


def kernel(node_features, edge_index, capacities, W0, b0, W1, b1):
    raise NotImplementedError("write your pallas kernel here")


# speedup vs baseline: 3.6132x; 3.6132x over previous
import functools

import jax
import jax.numpy as jnp
from jax import lax
from jax.experimental import pallas as pl
from jax.experimental.pallas import tpu as pltpu
from jax.experimental.pallas import tpu_sc as plsc

B = 2
N = 10000
E = 160000
F = 128
D0 = 129
D1 = 130
DP = 144
DM = 256
DE = 260
NX = 3

NC = 2
NS = 16
NW = NC * NS
EWR = E // NW
CH = 128
NJ = 40
EWP = NJ * CH
CHE = 64
NJE = EWP // CHE
CHX = 32
NJX = EWP // CHX
NRP = 10240
RPS = NRP // NS

_mesh = plsc.VectorSubcoreMesh(core_axis_name="c", subcore_axis_name="s")
_sc_params = pltpu.CompilerParams(needs_layout_passes=False)


def _worker_id():
    return lax.axis_index("s") * NC + lax.axis_index("c")


def _deg_body(col3, caps3, degp, colv, colv2, capv, zb, acc):
    c = lax.axis_index("c")
    s = lax.axis_index("s")
    wid = _worker_id()
    pltpu.sync_copy(col3.at[wid], colv)

    def bake(j, _):
        for g in range(CH // 16):
            colv2[j, pl.ds(g * 16, 16)] = colv[j, pl.ds(g * 16, 16)] + NRP
        return 0
    lax.fori_loop(0, NJ, bake, 0)

    def zbody(i, _):
        zb[pl.ds(i * 16, 16)] = jnp.zeros((16,), jnp.float32)
        return 0
    lax.fori_loop(0, CH // 16, zbody, 0)
    for q in range(2 * RPS // CH):
        pltpu.sync_copy(zb, acc.at[pl.ds(s * 2 * RPS + q * CH, CH)])
    plsc.subcore_barrier()

    for b in range(B):
        pltpu.sync_copy(caps3.at[b, wid], capv)
        cv = colv if b == 0 else colv2

        def jbody(j, _):
            pltpu.sync_copy(capv.at[j], acc.at[cv.at[j]], add=True)
            return 0
        lax.fori_loop(0, NJ, jbody, 0)
    plsc.subcore_barrier()
    for b in range(B):
        pltpu.sync_copy(acc.at[pl.ds(b * NRP + s * RPS, RPS)],
                        degp.at[c, b, pl.ds(s * RPS, RPS)])


_deg_call = pl.kernel(
    _deg_body,
    out_type=jax.ShapeDtypeStruct((NC, B, NRP), jnp.float32),
    mesh=_mesh,
    compiler_params=_sc_params,
    scratch_types=[
        pltpu.VMEM((NJ, CH), jnp.int32),
        pltpu.VMEM((NJ, CH), jnp.int32),
        pltpu.VMEM((NJ, CH), jnp.float32),
        pltpu.VMEM((CH,), jnp.float32),
        pltpu.VMEM_SHARED((2 * NRP,), jnp.float32),
    ],
)


def _agg_body(row3, col3, caps3, u2, tout, rowv, colv, capv, gbuf, acc, sem):
    c = lax.axis_index("c")
    s = lax.axis_index("s")
    wid = _worker_id()
    pltpu.sync_copy(col3.at[wid], colv)

    def zgbuf():
        def zb(r, _):
            for k in range(F // 16):
                gbuf[r, pl.ds(k * 16, 16)] = jnp.zeros((16,), jnp.float32)
            return 0
        lax.fori_loop(0, CH, zb, 0)

    for b in range(B):
        zgbuf()
        for q in range(RPS // CH):
            pltpu.sync_copy(gbuf, acc.at[pl.ds(s * RPS + q * CH, CH), :])
        plsc.subcore_barrier()
        pltpu.sync_copy(row3.at[wid], rowv)
        pltpu.sync_copy(caps3.at[b, wid], capv)

        def jbody(j, _):
            pltpu.async_copy(u2.at[b].at[rowv.at[j]], gbuf, sem).wait()

            def rbody(g, _):
                cap16 = capv[j, pl.ds(g * 16, 16)]
                for i in range(16):
                    w = cap16[i]
                    r = g * 16 + i
                    for k in range(F // 16):
                        gbuf[r, pl.ds(k * 16, 16)] = (
                            gbuf[r, pl.ds(k * 16, 16)] * w)
                return 0
            lax.fori_loop(0, CH // 16, rbody, 0)
            pltpu.sync_copy(gbuf, acc.at[colv.at[j]], add=True)
            return 0
        lax.fori_loop(0, NJ, jbody, 0)
        plsc.subcore_barrier()
        pltpu.sync_copy(acc.at[pl.ds(s * RPS, RPS), :],
                        tout.at[c, b, pl.ds(s * RPS, RPS), :])
        plsc.subcore_barrier()


_agg0_call = pl.kernel(
    _agg_body,
    out_type=jax.ShapeDtypeStruct((NC, B, NRP, F), jnp.float32),
    mesh=_mesh,
    compiler_params=_sc_params,
    scratch_types=[
        pltpu.VMEM((NJ, CH), jnp.int32),
        pltpu.VMEM((NJ, CH), jnp.int32),
        pltpu.VMEM((NJ, CH), jnp.float32),
        pltpu.VMEM((CH, F), jnp.float32),
        pltpu.VMEM_SHARED((NRP, F), jnp.float32),
        pltpu.SemaphoreType.DMA,
    ],
)

def _aggx_body(row3, col3, caps3, uxh, txout, rowv, colv, colv2, capv,
               zb, xg, xbuf, uxs, accx):
    c = lax.axis_index("c")
    s = lax.axis_index("s")
    wid = _worker_id()
    pltpu.sync_copy(col3.at[wid], colv)
    pltpu.sync_copy(row3.at[wid], rowv)

    def bake(j, _):
        for g in range(CH // 16):
            colv2[j, pl.ds(g * 16, 16)] = colv[j, pl.ds(g * 16, 16)] + NRP
        return 0
    lax.fori_loop(0, NJ, bake, 0)

    def zbody(i, _):
        zb[pl.ds(i * 16, 16)] = jnp.zeros((16,), jnp.float32)
        return 0
    lax.fori_loop(0, CH // 16, zbody, 0)
    for q in range(2 * RPS // CH):
        pltpu.sync_copy(zb, accx.at[pl.ds(s * 2 * RPS + q * CH, CH)])

    for b in range(B):
        pltpu.sync_copy(uxh.at[b, pl.ds(s * RPS, RPS)],
                        uxs.at[pl.ds(s * RPS, RPS)])
        plsc.subcore_barrier()
        pltpu.sync_copy(caps3.at[b, wid], capv)
        cv = colv if b == 0 else colv2

        def jbody(j, _):
            pltpu.sync_copy(uxs.at[rowv.at[j]], xg)

            def rbody(g, _):
                xbuf[pl.ds(g * 16, 16)] = (
                    xg[pl.ds(g * 16, 16)] * capv[j, pl.ds(g * 16, 16)])
                return 0
            lax.fori_loop(0, CH // 16, rbody, 0)
            pltpu.sync_copy(xbuf, accx.at[cv.at[j]], add=True)
            return 0
        lax.fori_loop(0, NJ, jbody, 0)
        plsc.subcore_barrier()
    for b in range(B):
        pltpu.sync_copy(accx.at[pl.ds(b * NRP + s * RPS, RPS)],
                        txout.at[c, b, pl.ds(s * RPS, RPS)])


_aggx_call = pl.kernel(
    _aggx_body,
    out_type=jax.ShapeDtypeStruct((NC, B, NRP), jnp.float32),
    mesh=_mesh,
    compiler_params=_sc_params,
    scratch_types=[
        pltpu.VMEM((NJ, CH), jnp.int32),
        pltpu.VMEM((NJ, CH), jnp.int32),
        pltpu.VMEM((NJ, CH), jnp.int32),
        pltpu.VMEM((NJ, CH), jnp.float32),
        pltpu.VMEM((CH,), jnp.float32),
        pltpu.VMEM((CH,), jnp.float32),
        pltpu.VMEM((CH,), jnp.float32),
        pltpu.VMEM_SHARED((NRP,), jnp.float32),
        pltpu.VMEM_SHARED((2 * NRP,), jnp.float32),
    ],
)


def _edge_body(row3, col3, caps3, emb2, exh, out, rowv, colv, capv,
               sbuf, dbuf, obuf, exv, sem1, sem2):
    wid = _worker_id()

    for b in range(B):
        pltpu.sync_copy(row3.at[wid], rowv)
        pltpu.sync_copy(col3.at[wid], colv)
        pltpu.sync_copy(caps3.at[b, wid], capv)
        for t in range(NX):
            pltpu.sync_copy(exh.at[b, t],
                            exv.at[pl.ds(t * (NRP // 128), NRP // 128), :])

        def compute(j):
            cp1 = pltpu.async_copy(emb2.at[b].at[rowv.at[j]], sbuf, sem1)
            cp2 = pltpu.async_copy(emb2.at[b].at[colv.at[j]], dbuf, sem2)
            cp1.wait()
            cp2.wait()

            def rbody(g, _):
                cap16 = capv[j, pl.ds(g * 16, 16)]
                s16 = rowv[j, pl.ds(g * 16, 16)]
                d16 = colv[j, pl.ds(g * 16, 16)]
                ridx = g * 16 + lax.broadcasted_iota(jnp.int32, (16,), 0)
                for t in range(NX):
                    sf = s16 + t * NRP
                    df = d16 + t * NRP
                    ex = (plsc.load_gather(exv, [sf >> 7, sf & 127])
                          + plsc.load_gather(exv, [df >> 7, df & 127]))
                    plsc.store_scatter(obuf, [ridx,
                                              jnp.full((16,), DM + t,
                                                       jnp.int32)], ex)
                plsc.store_scatter(obuf, [ridx,
                                          jnp.full((16,), DE - 1,
                                                   jnp.int32)], cap16)
                for i in range(16):
                    r = g * 16 + i
                    for k in range(DM // 16):
                        obuf[r, pl.ds(k * 16, 16)] = (
                            sbuf[r, pl.ds(k * 16, 16)]
                            + dbuf[r, pl.ds(k * 16, 16)])
                return 0
            lax.fori_loop(0, CHE // 16, rbody, 0)

        nfull = EWR // CHE
        rem = EWR - nfull * CHE

        def jbody(j, _):
            compute(j)
            pltpu.sync_copy(obuf, out.at[b, pl.ds(wid * EWR + j * CHE, CHE), :])
            return 0
        lax.fori_loop(0, nfull, jbody, 0)
        compute(nfull)
        pltpu.sync_copy(obuf.at[pl.ds(0, rem), :],
                        out.at[b, pl.ds(wid * EWR + nfull * CHE, rem), :])


_edge_call = pl.kernel(
    _edge_body,
    out_type=jax.ShapeDtypeStruct((B, E, DE), jnp.float32),
    mesh=_mesh,
    compiler_params=_sc_params,
    scratch_types=[
        pltpu.VMEM((NJE, CHE), jnp.int32),
        pltpu.VMEM((NJE, CHE), jnp.int32),
        pltpu.VMEM((NJE, CHE), jnp.float32),
        pltpu.VMEM((CHE, DM), jnp.float32),
        pltpu.VMEM((CHE, DM), jnp.float32),
        pltpu.VMEM((CHE, DE), jnp.float32),
        pltpu.VMEM((NRP * NX // 128, 128), jnp.float32),
        pltpu.SemaphoreType.DMA,
        pltpu.SemaphoreType.DMA,
    ],
)


def _tc_z_body(nf_ref, degp_ref, z_ref, dinv_ref):
    deg = degp_ref[0] + degp_ref[1] + 1.0
    dinv = jnp.where(deg > 0, lax.rsqrt(jnp.maximum(deg, 1e-12)), 0.0)
    dinv_ref[...] = dinv
    for b in range(B):
        z_ref[b, :N] = nf_ref[b] * dinv[b, :N][:, None]
        z_ref[b, N:] = jnp.zeros((NRP - N, F), jnp.float32)


def _tc_z(nf, degp):
    return pl.pallas_call(
        _tc_z_body,
        out_shape=(jax.ShapeDtypeStruct((B, NRP, F), jnp.float32),
                   jax.ShapeDtypeStruct((B, NRP), jnp.float32)),
    )(nf, degp)


RB = 2048
GN = NRP // RB


def _tc_layer1_body(t_ref, z_ref, dinv_ref, w0_ref, b0_ref,
                    h0_ref, u1m_ref, u1x_ref):
    for b in range(B):
        dv = dinv_ref[b][:, None]
        v = (t_ref[0, b] + t_ref[1, b] + z_ref[b]) * dv
        pre = jnp.dot(v, w0_ref[...],
                      preferred_element_type=jnp.float32) + b0_ref[...]
        h0 = jnp.where(pre > 0, pre, 0.02 * pre)
        h0_ref[b] = h0
        u1 = h0 * dv
        u1m_ref[b] = u1[:, :F]
        u1x_ref[b] = u1[:, F]


def _tc_layer1(t0, z, dinv, w0p, b0p):
    return pl.pallas_call(
        _tc_layer1_body,
        grid=(GN,),
        in_specs=[
            pl.BlockSpec((NC, B, RB, F), lambda i: (0, 0, i, 0)),
            pl.BlockSpec((B, RB, F), lambda i: (0, i, 0)),
            pl.BlockSpec((B, RB), lambda i: (0, i)),
            pl.BlockSpec((F, DP), lambda i: (0, 0)),
            pl.BlockSpec((1, DP), lambda i: (0, 0)),
        ],
        out_specs=(pl.BlockSpec((B, RB, DP), lambda i: (0, i, 0)),
                   pl.BlockSpec((B, RB, F), lambda i: (0, i, 0)),
                   pl.BlockSpec((B, RB), lambda i: (0, i))),
        out_shape=(jax.ShapeDtypeStruct((B, NRP, DP), jnp.float32),
                   jax.ShapeDtypeStruct((B, NRP, F), jnp.float32),
                   jax.ShapeDtypeStruct((B, NRP), jnp.float32)),
    )(t0, z, dinv, w0p, b0p)


def _tc_pack_body(tm_ref, tx_ref, u1m_ref, u1x_ref, dinv_ref, w1_ref, b1_ref,
                  h0_ref, emb_ref, exh_ref):
    for b in range(B):
        dv = dinv_ref[b][:, None]
        m = (tm_ref[0, b] + tm_ref[1, b] + u1m_ref[b]) * dv
        x = (tx_ref[0, b] + tx_ref[1, b] + u1x_ref[b]) * dinv_ref[b]
        v = jnp.concatenate(
            [m, x[:, None], jnp.zeros((RB, DP - F - 1), jnp.float32)], axis=1)
        pre = jnp.dot(v, w1_ref[...],
                      preferred_element_type=jnp.float32) + b1_ref[...]
        h1 = jnp.where(pre > 0, pre, 0.02 * pre)
        emb_ref[b] = jnp.concatenate(
            [h0_ref[b][:, :D0], h1[:, :DM - D0]], axis=1)
        for t in range(NX):
            exh_ref[b, t] = h1[:, DM - D0 + t]


def _tc_pack(t1m, t1x, u1m, u1x, dinv, w1p, b1p, h0):
    return pl.pallas_call(
        _tc_pack_body,
        grid=(GN,),
        in_specs=[
            pl.BlockSpec((NC, B, RB, F), lambda i: (0, 0, i, 0)),
            pl.BlockSpec((NC, B, RB), lambda i: (0, 0, i)),
            pl.BlockSpec((B, RB, F), lambda i: (0, i, 0)),
            pl.BlockSpec((B, RB), lambda i: (0, i)),
            pl.BlockSpec((B, RB), lambda i: (0, i)),
            pl.BlockSpec((DP, DP), lambda i: (0, 0)),
            pl.BlockSpec((1, DP), lambda i: (0, 0)),
            pl.BlockSpec((B, RB, DP), lambda i: (0, i, 0)),
        ],
        out_specs=(pl.BlockSpec((B, RB, DM), lambda i: (0, i, 0)),
                   pl.BlockSpec((B, NX, RB), lambda i: (0, 0, i))),
        out_shape=(jax.ShapeDtypeStruct((B, NRP, DM), jnp.float32),
                   jax.ShapeDtypeStruct((B, NX, NRP), jnp.float32)),
    )(t1m, t1x, u1m, u1x, dinv, w1p, b1p, h0)


@jax.jit
def kernel(node_features, edge_index, capacities, W0, b0, W1, b1):
    row = edge_index[0]
    col = edge_index[1]

    def tile3(x, ch):
        x = x.reshape(NW, EWR)
        x = jnp.pad(x, ((0, 0), (0, EWP - EWR)))
        return x.reshape(NW, EWP // ch, ch)

    col3 = tile3(col, CH)
    row3 = tile3(row, CH)
    rowe3 = tile3(row, CHE)
    cole3 = tile3(col, CHE)
    capsp = jnp.pad(capacities.reshape(B, NW, EWR),
                    ((0, 0), (0, 0), (0, EWP - EWR)))
    caps3 = capsp.reshape(B, NW, NJ, CH)
    capse3 = capsp.reshape(B, NW, NJE, CHE)

    w0p = jnp.pad(W0, ((0, 0), (0, DP - D0)))
    b0p = jnp.pad(b0, (0, DP - D0))[None, :]
    w1p = jnp.pad(W1, ((0, DP - D0), (0, DP - D1)))
    b1p = jnp.pad(b1, (0, DP - D1))[None, :]

    degp = _deg_call(col3, caps3)
    z, dinv = _tc_z(node_features, degp)
    t0 = _agg0_call(row3, col3, caps3, z)
    h0, u1m, u1x = _tc_layer1(t0, z, dinv, w0p, b0p)
    t1m = _agg0_call(row3, col3, caps3, u1m)
    t1x = _aggx_call(row3, col3, caps3, u1x)
    emb, exh = _tc_pack(t1m, t1x, u1m, u1x, dinv, w1p, b1p, h0)
    out = _edge_call(rowe3, cole3, capse3, emb,
                     exh.reshape(B, NX, NRP // 128, 128))
    return out
